# baseline (speedup 1.0000x reference)
import functools

import jax
import jax.numpy as jnp
import numpy as np
from jax import lax
from jax.experimental import pallas as pl
from jax.experimental.pallas import tpu as pltpu
from jax.experimental.pallas import tpu_sc as plsc

_B = 64
_N = 8192
_K = 64
_L = 16
_NV = _N // _L
_NC = 2
_NS = 16
_RPW = _B // (_NC * _NS)
_UNROLL = 4

_SIGN = np.int32(-2147483648)
_MANT = np.int32(0x7FFFFFFF)
_ONE = np.int32(1)
_CAND = _N + 4 * _L


def _monotone_keys(x):
    b = lax.bitcast_convert_type(x, jnp.int32)
    return b ^ ((b >> 31) & _MANT)


def _popcnt(m):
    return plsc.all_reduce_population_count(m)[0]


def _topk_body(logits_hbm, out_hbm, rows_v, out_v, mx_v, gi_v, ck_v, ci_v,
               sem):
    cid = lax.axis_index("c")
    sid = lax.axis_index("s")
    wid = sid * _NC + cid
    base = wid * _RPW
    pltpu.sync_copy(logits_hbm.at[pl.ds(base, _RPW)], rows_v)

    zeros = jnp.zeros((_L,), jnp.int32)
    zf = jnp.zeros((_L,), jnp.float32)
    minv = jnp.full((_L,), _SIGN, jnp.int32)

    def gmax_body(c, carry):
        def inner(i, ms):
            m0, m1 = ms
            for u in range(_UNROLL):
                sl = pl.ds((c * 32 + i * _UNROLL + u) * _L, _L)
                m0 = jnp.maximum(m0, _monotone_keys(rows_v[0, sl]))
                m1 = jnp.maximum(m1, _monotone_keys(rows_v[1, sl]))
                out_v[0, sl] = zf
                out_v[1, sl] = zf
            return m0, m1

        m0, m1 = lax.fori_loop(0, 32 // _UNROLL, inner, (minv, minv))
        mx_v[pl.ds(c * _L, _L)] = m0
        mx_v[pl.ds(256 + c * _L, _L)] = m1
        return carry

    lax.fori_loop(0, 16, gmax_body, np.int32(0))

    def mbit_body(j, tbs):
        tb0, tb1 = tbs
        bit = _ONE << (np.int32(31) - j)
        c0s = (tb0 | bit) ^ _SIGN
        c1s = (tb1 | bit) ^ _SIGN

        def cnt_body(i, accs):
            a0, a1 = accs
            for u in range(_UNROLL):
                sl = pl.ds((i * _UNROLL + u) * _L, _L)
                slb = pl.ds(256 + (i * _UNROLL + u) * _L, _L)
                a0 = a0 + (mx_v[sl] >= c0s).astype(jnp.int32)
                a1 = a1 + (mx_v[slb] >= c1s).astype(jnp.int32)
            return a0, a1

        a0, a1 = lax.fori_loop(0, 16 // _UNROLL, cnt_body, (zeros, zeros))
        tb0 = jnp.where(jnp.sum(a0) >= _K, tb0 | bit, tb0)
        tb1 = jnp.where(jnp.sum(a1) >= _K, tb1 | bit, tb1)
        return tb0, tb1

    tb0, tb1 = lax.fori_loop(0, 16, mbit_body,
                             (np.int32(0), np.int32(0)))

    copies = []
    for r, tb in ((0, tb0), (1, tb1)):
        ts = tb ^ _SIGN

        iota = jnp.arange(_L, dtype=jnp.int32)

        def gl_body(i, goff, r=r, ts=ts):
            m = mx_v[pl.ds(r * 256 + i * _L, _L)] >= ts
            plsc.store_compressed(gi_v.at[pl.ds(goff, _L)],
                                  iota + i * _L, mask=m)
            return goff + _popcnt(m)

        ng = lax.fori_loop(0, 16, gl_body, np.int32(0))

        rix = jnp.full((_L,), np.int32(r), jnp.int32)
        gi_v[pl.ds(ng, _L)] = zeros
        nb = (ng + _L - 1) // _L

        def comp_body(b, off, r=r, ts=ts, rix=rix, ng=ng):
            gv = gi_v[pl.ds(b * _L, _L)]
            laneok = (iota + b * _L) < ng
            basep = (gv >> 4) * 512 + (gv & 15)
            for i in range(32):
                pos = basep + i * _L
                s = _monotone_keys(plsc.load_gather(rows_v, [rix, pos]))
                m = (s >= ts) & laneok
                plsc.store_compressed(ck_v.at[pl.ds(off, _L)], s, mask=m)
                plsc.store_compressed(ci_v.at[pl.ds(off, _L)], pos, mask=m)
                off = off + _popcnt(m)
            return off

        nc = lax.fori_loop(0, nb, comp_body, np.int32(0))
        for u in range(_UNROLL):
            ck_v[pl.ds(nc + u * _L, _L)] = jnp.full((_L,), _SIGN, jnp.int32)
            ci_v[pl.ds(nc + u * _L, _L)] = zeros
        nv2 = (nc + 4 * _L - 1) // (4 * _L)

        def bit2_body(j, tb, nv2=nv2):
            cb = tb | (_ONE << (np.int32(31) - j))
            cs = cb ^ _SIGN

            def cnt_body(i, acc):
                for u in range(_UNROLL):
                    sl = pl.ds((i * _UNROLL + u) * _L, _L)
                    acc = acc + plsc.all_reduce_population_count(
                        ck_v[sl] >= cs)
                return acc

            acc = lax.fori_loop(0, nv2, cnt_body, zeros)
            return jnp.where(acc[0] >= _K, cb, tb)

        tb = lax.fori_loop(0, 32, bit2_body, np.int32(0))
        vstar = tb ^ _SIGN

        def cnt2_body(i, accs, vstar=vstar):
            ag, ae = accs
            for u in range(_UNROLL):
                sl = pl.ds((i * _UNROLL + u) * _L, _L)
                s = ck_v[sl]
                ag = ag + plsc.all_reduce_population_count(s > vstar)
                ae = ae + plsc.all_reduce_population_count(s == vstar)
            return ag, ae

        ag, ae = lax.fori_loop(0, nv2, cnt2_body, (zeros, zeros))
        need = _K - ag[0]
        extra = ae[0] - need
        m1 = extra + _ONE

        def ibit_body(jj, t, vstar=vstar, nv2=nv2, m1=m1):
            cand = t | (_ONE << (np.int32(12) - jj))

            def cnt_body(i, acc):
                for u in range(_UNROLL):
                    sl = pl.ds((i * _UNROLL + u) * _L, _L)
                    acc = acc + plsc.all_reduce_population_count(
                        (ck_v[sl] == vstar) & (ci_v[sl] >= cand))
                return acc

            acc = lax.fori_loop(0, nv2, cnt_body, zeros)
            return jnp.where(acc[0] >= m1, cand, t)

        icut = lax.fori_loop(
            0, jnp.where(extra > 0, np.int32(13), np.int32(0)),
            ibit_body, jnp.where(extra > 0, np.int32(0), np.int32(8191)))
        nv2s = (nc + _L - 1) // _L

        def sel_body(i, c, r=r, vstar=vstar, icut=icut, rix=rix):
            sl = pl.ds(i * _L, _L)
            s = ck_v[sl]
            idx = ci_v[sl]
            sel = (s > vstar) | ((s == vstar) & (idx <= icut))
            plsc.store_scatter(out_v, [rix, idx],
                               jnp.ones((_L,), jnp.float32), mask=sel)
            return c

        lax.fori_loop(0, nv2s, sel_body, np.int32(0))
        copies.append(pltpu.async_copy(
            out_v.at[pl.ds(r, 1)], out_hbm.at[pl.ds(base + r, 1)], sem))

    for cp in copies:
        cp.wait()


@functools.partial(
    pl.kernel,
    out_type=jax.ShapeDtypeStruct((_B, _N), jnp.float32),
    mesh=plsc.VectorSubcoreMesh(
        core_axis_name="c", subcore_axis_name="s",
        num_cores=_NC, num_subcores=_NS),
    scratch_types=[
        pltpu.VMEM((_RPW, _N), jnp.float32),
        pltpu.VMEM((_RPW, _N), jnp.float32),
        pltpu.VMEM((512,), jnp.int32),
        pltpu.VMEM((272,), jnp.int32),
        pltpu.VMEM((_CAND,), jnp.int32),
        pltpu.VMEM((_CAND,), jnp.int32),
        pltpu.SemaphoreType.DMA,
    ],
    compiler_params=pltpu.CompilerParams(needs_layout_passes=False),
)
def _topk_onehot(logits_hbm, out_hbm, rows_v, out_v, mx_v, gi_v, ck_v, ci_v,
                 sem):
    _topk_body(logits_hbm, out_hbm, rows_v, out_v, mx_v, gi_v, ck_v, ci_v,
               sem)


def kernel(logits, k):
    del k
    return _topk_onehot(logits)


# speedup vs baseline: 1.0250x; 1.0250x over previous
import functools

import jax
import jax.numpy as jnp
import numpy as np
from jax import lax
from jax.experimental import pallas as pl
from jax.experimental.pallas import tpu as pltpu
from jax.experimental.pallas import tpu_sc as plsc

_B = 64
_N = 8192
_K = 64
_L = 16
_NV = _N // _L
_NC = 2
_NS = 16
_RPW = _B // (_NC * _NS)
_UNROLL = 4

_SIGN = np.int32(-2147483648)
_MANT = np.int32(0x7FFFFFFF)
_ONE = np.int32(1)
_CAND = _N + 4 * _L
_GI = 272


def _monotone_keys(x):
    b = lax.bitcast_convert_type(x, jnp.int32)
    return b ^ ((b >> 31) & _MANT)


def _popcnt(m):
    return plsc.all_reduce_population_count(m)[0]


def _topk_body(logits_hbm, out_hbm, rows_v, out_v, mx_v, gi_v, ck_v, ci_v,
               sem):
    cid = lax.axis_index("c")
    sid = lax.axis_index("s")
    wid = sid * _NC + cid
    base = wid * _RPW
    pltpu.sync_copy(logits_hbm.at[pl.ds(base, _RPW)], rows_v)

    zeros = jnp.zeros((_L,), jnp.int32)
    zf = jnp.zeros((_L,), jnp.float32)
    minv = jnp.full((_L,), _SIGN, jnp.int32)

    def gmax_body(c, carry):
        def inner(i, ms):
            m0, m1 = ms
            for u in range(_UNROLL):
                sl = pl.ds((c * 32 + i * _UNROLL + u) * _L, _L)
                m0 = jnp.maximum(m0, _monotone_keys(rows_v[0, sl]))
                m1 = jnp.maximum(m1, _monotone_keys(rows_v[1, sl]))
                out_v[0, sl] = zf
                out_v[1, sl] = zf
            return m0, m1

        m0, m1 = lax.fori_loop(0, 32 // _UNROLL, inner, (minv, minv))
        mx_v[pl.ds(c * _L, _L)] = m0
        mx_v[pl.ds(256 + c * _L, _L)] = m1
        return carry

    lax.fori_loop(0, 16, gmax_body, np.int32(0))

    def mbit_body(j, tbs):
        tb0, tb1 = tbs
        bit = _ONE << (np.int32(31) - j)
        c0s = (tb0 | bit) ^ _SIGN
        c1s = (tb1 | bit) ^ _SIGN

        def cnt_body(i, accs):
            a0, a1 = accs
            for u in range(_UNROLL):
                sl = pl.ds((i * _UNROLL + u) * _L, _L)
                slb = pl.ds(256 + (i * _UNROLL + u) * _L, _L)
                a0 = a0 + (mx_v[sl] >= c0s).astype(jnp.int32)
                a1 = a1 + (mx_v[slb] >= c1s).astype(jnp.int32)
            return a0, a1

        a0, a1 = lax.fori_loop(0, 16 // _UNROLL, cnt_body, (zeros, zeros))
        tb0 = jnp.where(jnp.sum(a0) >= _K, tb0 | bit, tb0)
        tb1 = jnp.where(jnp.sum(a1) >= _K, tb1 | bit, tb1)
        return tb0, tb1

    tb0, tb1 = lax.fori_loop(0, 16, mbit_body,
                             (np.int32(0), np.int32(0)))

    ts0 = tb0 ^ _SIGN
    ts1 = tb1 ^ _SIGN
    iota = jnp.arange(_L, dtype=jnp.int32)

    def gl_body(i, goffs):
        g0, g1 = goffs
        gid = iota + i * _L
        m0 = mx_v[pl.ds(i * _L, _L)] >= ts0
        m1 = mx_v[pl.ds(256 + i * _L, _L)] >= ts1
        plsc.store_compressed(gi_v.at[pl.ds(g0, _L)], gid, mask=m0)
        plsc.store_compressed(gi_v.at[pl.ds(_GI + g1, _L)], gid, mask=m1)
        return g0 + _popcnt(m0), g1 + _popcnt(m1)

    ng0, ng1 = lax.fori_loop(0, 16, gl_body, (np.int32(0), np.int32(0)))
    gi_v[pl.ds(ng0, _L)] = zeros
    gi_v[pl.ds(_GI + ng1, _L)] = zeros

    rix0 = jnp.full((_L,), np.int32(0), jnp.int32)
    rix1 = jnp.full((_L,), np.int32(1), jnp.int32)
    nb0 = (ng0 + _L - 1) // _L
    nb1 = (ng1 + _L - 1) // _L
    nb = jnp.maximum(nb0, nb1)

    def comp_body(b, offs):
        o0, o1 = offs
        gv0 = gi_v[pl.ds(b * _L, _L)] & np.int32(255)
        gv1 = gi_v[pl.ds(_GI + b * _L, _L)] & np.int32(255)
        ok0 = (iota + b * _L) < ng0
        ok1 = (iota + b * _L) < ng1
        bp0 = (gv0 >> 4) * 512 + (gv0 & 15)
        bp1 = (gv1 >> 4) * 512 + (gv1 & 15)
        for i in range(32):
            p0 = bp0 + i * _L
            s0 = _monotone_keys(plsc.load_gather(rows_v, [rix0, p0]))
            m0 = (s0 >= ts0) & ok0
            plsc.store_compressed(ck_v.at[pl.ds(o0, _L)], s0, mask=m0)
            plsc.store_compressed(ci_v.at[pl.ds(o0, _L)], p0, mask=m0)
            o0 = o0 + _popcnt(m0)
            p1 = bp1 + i * _L
            s1 = _monotone_keys(plsc.load_gather(rows_v, [rix1, p1]))
            m1 = (s1 >= ts1) & ok1
            plsc.store_compressed(ck_v.at[pl.ds(_CAND + o1, _L)], s1,
                                  mask=m1)
            plsc.store_compressed(ci_v.at[pl.ds(_CAND + o1, _L)], p1,
                                  mask=m1)
            o1 = o1 + _popcnt(m1)
        return o0, o1

    nc0, nc1 = lax.fori_loop(0, nb, comp_body, (np.int32(0), np.int32(0)))

    copies = []
    for r, ts, nc, rix in ((0, ts0, nc0, rix0), (1, ts1, nc1, rix1)):
        rb = r * _CAND
        for u in range(_UNROLL):
            ck_v[pl.ds(rb + nc + u * _L, _L)] = jnp.full(
                (_L,), _SIGN, jnp.int32)
            ci_v[pl.ds(rb + nc + u * _L, _L)] = zeros
        nv2 = (nc + 4 * _L - 1) // (4 * _L)

        def bit2_body(j, tb, nv2=nv2, rb=rb):
            cb = tb | (_ONE << (np.int32(31) - j))
            cs = cb ^ _SIGN

            def cnt_body(i, acc):
                for u in range(_UNROLL):
                    sl = pl.ds(rb + (i * _UNROLL + u) * _L, _L)
                    acc = acc + plsc.all_reduce_population_count(
                        ck_v[sl] >= cs)
                return acc

            acc = lax.fori_loop(0, nv2, cnt_body, zeros)
            return jnp.where(acc[0] >= _K, cb, tb)

        tb = lax.fori_loop(0, 32, bit2_body, np.int32(0))
        vstar = tb ^ _SIGN

        def cnt2_body(i, accs, vstar=vstar, rb=rb):
            ag, ae = accs
            for u in range(_UNROLL):
                sl = pl.ds(rb + (i * _UNROLL + u) * _L, _L)
                s = ck_v[sl]
                ag = ag + plsc.all_reduce_population_count(s > vstar)
                ae = ae + plsc.all_reduce_population_count(s == vstar)
            return ag, ae

        ag, ae = lax.fori_loop(0, nv2, cnt2_body, (zeros, zeros))
        need = _K - ag[0]
        extra = ae[0] - need
        m1 = extra + _ONE

        def ibit_body(jj, t, vstar=vstar, nv2=nv2, m1=m1, rb=rb):
            cand = t | (_ONE << (np.int32(12) - jj))

            def cnt_body(i, acc):
                for u in range(_UNROLL):
                    sl = pl.ds(rb + (i * _UNROLL + u) * _L, _L)
                    acc = acc + plsc.all_reduce_population_count(
                        (ck_v[sl] == vstar) & (ci_v[sl] >= cand))
                return acc

            acc = lax.fori_loop(0, nv2, cnt_body, zeros)
            return jnp.where(acc[0] >= m1, cand, t)

        icut = lax.fori_loop(
            0, jnp.where(extra > 0, np.int32(13), np.int32(0)),
            ibit_body, jnp.where(extra > 0, np.int32(0), np.int32(8191)))
        nv2s = (nc + _L - 1) // _L

        def sel_body(i, c, vstar=vstar, icut=icut, rix=rix, rb=rb):
            sl = pl.ds(rb + i * _L, _L)
            s = ck_v[sl]
            idx = ci_v[sl]
            sel = (s > vstar) | ((s == vstar) & (idx <= icut))
            plsc.store_scatter(out_v, [rix, idx],
                               jnp.ones((_L,), jnp.float32), mask=sel)
            return c

        lax.fori_loop(0, nv2s, sel_body, np.int32(0))
        copies.append(pltpu.async_copy(
            out_v.at[pl.ds(r, 1)], out_hbm.at[pl.ds(base + r, 1)], sem))

    for cp in copies:
        cp.wait()


@functools.partial(
    pl.kernel,
    out_type=jax.ShapeDtypeStruct((_B, _N), jnp.float32),
    mesh=plsc.VectorSubcoreMesh(
        core_axis_name="c", subcore_axis_name="s",
        num_cores=_NC, num_subcores=_NS),
    scratch_types=[
        pltpu.VMEM((_RPW, _N), jnp.float32),
        pltpu.VMEM((_RPW, _N), jnp.float32),
        pltpu.VMEM((512,), jnp.int32),
        pltpu.VMEM((2 * _GI,), jnp.int32),
        pltpu.VMEM((2 * _CAND,), jnp.int32),
        pltpu.VMEM((2 * _CAND,), jnp.int32),
        pltpu.SemaphoreType.DMA,
    ],
    compiler_params=pltpu.CompilerParams(needs_layout_passes=False),
)
def _topk_onehot(logits_hbm, out_hbm, rows_v, out_v, mx_v, gi_v, ck_v, ci_v,
                 sem):
    _topk_body(logits_hbm, out_hbm, rows_v, out_v, mx_v, gi_v, ck_v, ci_v,
               sem)


def kernel(logits, k):
    del k
    return _topk_onehot(logits)


# speedup vs baseline: 1.0490x; 1.0234x over previous
import functools

import jax
import jax.numpy as jnp
import numpy as np
from jax import lax
from jax.experimental import pallas as pl
from jax.experimental.pallas import tpu as pltpu
from jax.experimental.pallas import tpu_sc as plsc

_B = 64
_N = 8192
_K = 64
_L = 16
_NV = _N // _L
_NC = 2
_NS = 16
_RPW = _B // (_NC * _NS)
_UNROLL = 4

_SIGN = np.int32(-2147483648)
_MANT = np.int32(0x7FFFFFFF)
_ONE = np.int32(1)
_CAND = _N + 4 * _L
_GI = 272


def _monotone_keys(x):
    b = lax.bitcast_convert_type(x, jnp.int32)
    return b ^ ((b >> 31) & _MANT)


def _popcnt(m):
    return plsc.all_reduce_population_count(m)[0]


def _topk_body(logits_hbm, out_hbm, rows_v, out_v, mx_v, gi_v, ck_v, ci_v,
               sem):
    cid = lax.axis_index("c")
    sid = lax.axis_index("s")
    wid = sid * _NC + cid
    base = wid * _RPW
    pltpu.sync_copy(logits_hbm.at[pl.ds(base, _RPW)], rows_v)

    zeros = jnp.zeros((_L,), jnp.int32)
    zf = jnp.zeros((_L,), jnp.float32)
    minv = jnp.full((_L,), _SIGN, jnp.int32)

    def gmax_body(c, carry):
        def inner(i, ms):
            m0, m1 = ms
            for u in range(_UNROLL):
                sl = pl.ds((c * 32 + i * _UNROLL + u) * _L, _L)
                m0 = jnp.maximum(m0, _monotone_keys(rows_v[0, sl]))
                m1 = jnp.maximum(m1, _monotone_keys(rows_v[1, sl]))
                out_v[0, sl] = zf
                out_v[1, sl] = zf
            return m0, m1

        m0, m1 = lax.fori_loop(0, 32 // _UNROLL, inner, (minv, minv))
        mx_v[pl.ds(c * _L, _L)] = m0
        mx_v[pl.ds(256 + c * _L, _L)] = m1
        return carry

    lax.fori_loop(0, 16, gmax_body, np.int32(0))

    def mbit_body(j, tbs):
        tb0, tb1 = tbs
        bit = _ONE << (np.int32(31) - j)
        c0s = (tb0 | bit) ^ _SIGN
        c1s = (tb1 | bit) ^ _SIGN

        def cnt_body(i, accs):
            a0, a1 = accs
            for u in range(_UNROLL):
                sl = pl.ds((i * _UNROLL + u) * _L, _L)
                slb = pl.ds(256 + (i * _UNROLL + u) * _L, _L)
                a0 = a0 + (mx_v[sl] >= c0s).astype(jnp.int32)
                a1 = a1 + (mx_v[slb] >= c1s).astype(jnp.int32)
            return a0, a1

        a0, a1 = lax.fori_loop(0, 16 // _UNROLL, cnt_body, (zeros, zeros))
        tb0 = jnp.where(jnp.sum(a0) >= _K, tb0 | bit, tb0)
        tb1 = jnp.where(jnp.sum(a1) >= _K, tb1 | bit, tb1)
        return tb0, tb1

    tb0, tb1 = lax.fori_loop(0, 16, mbit_body,
                             (np.int32(0), np.int32(0)))

    ts0 = tb0 ^ _SIGN
    ts1 = tb1 ^ _SIGN
    iota = jnp.arange(_L, dtype=jnp.int32)

    def gl_body(i, goffs):
        g0, g1 = goffs
        gid = iota + i * _L
        m0 = mx_v[pl.ds(i * _L, _L)] >= ts0
        m1 = mx_v[pl.ds(256 + i * _L, _L)] >= ts1
        plsc.store_compressed(gi_v.at[pl.ds(g0, _L)], gid, mask=m0)
        plsc.store_compressed(gi_v.at[pl.ds(_GI + g1, _L)], gid, mask=m1)
        return g0 + _popcnt(m0), g1 + _popcnt(m1)

    ng0, ng1 = lax.fori_loop(0, 16, gl_body, (np.int32(0), np.int32(0)))
    gi_v[pl.ds(ng0, _L)] = zeros
    gi_v[pl.ds(_GI + ng1, _L)] = zeros

    rix0 = jnp.full((_L,), np.int32(0), jnp.int32)
    rix1 = jnp.full((_L,), np.int32(1), jnp.int32)
    nb0 = (ng0 + _L - 1) // _L
    nb1 = (ng1 + _L - 1) // _L
    nb = jnp.maximum(nb0, nb1)

    def comp_body(b, offs):
        o0, o1 = offs
        gv0 = gi_v[pl.ds(b * _L, _L)] & np.int32(255)
        gv1 = gi_v[pl.ds(_GI + b * _L, _L)] & np.int32(255)
        ok0 = (iota + b * _L) < ng0
        ok1 = (iota + b * _L) < ng1
        bp0 = (gv0 >> 4) * 512 + (gv0 & 15)
        bp1 = (gv1 >> 4) * 512 + (gv1 & 15)
        for i in range(32):
            p0 = bp0 + i * _L
            s0 = _monotone_keys(plsc.load_gather(rows_v, [rix0, p0]))
            m0 = (s0 >= ts0) & ok0
            plsc.store_compressed(ck_v.at[pl.ds(o0, _L)], s0, mask=m0)
            plsc.store_compressed(ci_v.at[pl.ds(o0, _L)], p0, mask=m0)
            o0 = o0 + _popcnt(m0)
            p1 = bp1 + i * _L
            s1 = _monotone_keys(plsc.load_gather(rows_v, [rix1, p1]))
            m1 = (s1 >= ts1) & ok1
            plsc.store_compressed(ck_v.at[pl.ds(_CAND + o1, _L)], s1,
                                  mask=m1)
            plsc.store_compressed(ci_v.at[pl.ds(_CAND + o1, _L)], p1,
                                  mask=m1)
            o1 = o1 + _popcnt(m1)
        return o0, o1

    nc0, nc1 = lax.fori_loop(0, nb, comp_body, (np.int32(0), np.int32(0)))

    nv2 = (jnp.maximum(nc0, nc1) + 4 * _L - 1) // (4 * _L)

    def lanes(i, u):
        return iota + (i * _UNROLL + u) * _L

    def bit2_body(j, tbs):
        t0, t1 = tbs
        bit = _ONE << (np.int32(31) - j)
        cb0 = t0 | bit
        cb1 = t1 | bit
        cs0 = cb0 ^ _SIGN
        cs1 = cb1 ^ _SIGN

        def cnt_body(i, accs):
            a0, a1 = accs
            for u in range(_UNROLL):
                lv = lanes(i, u)
                sl0 = pl.ds((i * _UNROLL + u) * _L, _L)
                sl1 = pl.ds(_CAND + (i * _UNROLL + u) * _L, _L)
                a0 = a0 + plsc.all_reduce_population_count(
                    (ck_v[sl0] >= cs0) & (lv < nc0))
                a1 = a1 + plsc.all_reduce_population_count(
                    (ck_v[sl1] >= cs1) & (lv < nc1))
            return a0, a1

        a0, a1 = lax.fori_loop(0, nv2, cnt_body, (zeros, zeros))
        t0 = jnp.where(a0[0] >= _K, cb0, t0)
        t1 = jnp.where(a1[0] >= _K, cb1, t1)
        return t0, t1

    t0, t1 = lax.fori_loop(0, 32, bit2_body, (np.int32(0), np.int32(0)))
    vs0 = t0 ^ _SIGN
    vs1 = t1 ^ _SIGN

    def cnt2_body(i, accs):
        g0, e0, g1, e1 = accs
        for u in range(_UNROLL):
            lv = lanes(i, u)
            s0 = ck_v[pl.ds((i * _UNROLL + u) * _L, _L)]
            s1 = ck_v[pl.ds(_CAND + (i * _UNROLL + u) * _L, _L)]
            ok0 = lv < nc0
            ok1 = lv < nc1
            g0 = g0 + plsc.all_reduce_population_count((s0 > vs0) & ok0)
            e0 = e0 + plsc.all_reduce_population_count((s0 == vs0) & ok0)
            g1 = g1 + plsc.all_reduce_population_count((s1 > vs1) & ok1)
            e1 = e1 + plsc.all_reduce_population_count((s1 == vs1) & ok1)
        return g0, e0, g1, e1

    g0, e0, g1, e1 = lax.fori_loop(0, nv2, cnt2_body,
                                   (zeros, zeros, zeros, zeros))
    need0 = _K - g0[0]
    need1 = _K - g1[0]
    extra0 = e0[0] - need0
    extra1 = e1[0] - need1
    m10 = extra0 + _ONE
    m11 = extra1 + _ONE

    def ibit_body(jj, ts_):
        i0, i1 = ts_
        bit = _ONE << (np.int32(12) - jj)
        c0 = i0 | bit
        c1 = i1 | bit

        def cnt_body(i, accs):
            a0, a1 = accs
            for u in range(_UNROLL):
                lv = lanes(i, u)
                sl0 = pl.ds((i * _UNROLL + u) * _L, _L)
                sl1 = pl.ds(_CAND + (i * _UNROLL + u) * _L, _L)
                a0 = a0 + plsc.all_reduce_population_count(
                    (ck_v[sl0] == vs0) & (ci_v[sl0] >= c0) & (lv < nc0))
                a1 = a1 + plsc.all_reduce_population_count(
                    (ck_v[sl1] == vs1) & (ci_v[sl1] >= c1) & (lv < nc1))
            return a0, a1

        a0, a1 = lax.fori_loop(0, nv2, cnt_body, (zeros, zeros))
        i0n = jnp.where(a0[0] >= m10, c0, i0)
        i1n = jnp.where(a1[0] >= m11, c1, i1)
        return (jnp.where(extra0 > 0, i0n, np.int32(8191)),
                jnp.where(extra1 > 0, i1n, np.int32(8191)))

    anyx = (extra0 > 0) | (extra1 > 0)
    ic0, ic1 = lax.fori_loop(
        0, jnp.where(anyx, np.int32(13), np.int32(0)), ibit_body,
        (jnp.where(extra0 > 0, np.int32(0), np.int32(8191)),
         jnp.where(extra1 > 0, np.int32(0), np.int32(8191))))

    onesf = jnp.ones((_L,), jnp.float32)
    nv2s = (jnp.maximum(nc0, nc1) + _L - 1) // _L

    def sel_body(i, c):
        lv = iota + i * _L
        sl0 = pl.ds(i * _L, _L)
        sl1 = pl.ds(_CAND + i * _L, _L)
        s0 = ck_v[sl0]
        s1 = ck_v[sl1]
        id0 = ci_v[sl0]
        id1 = ci_v[sl1]
        s0m = ((s0 > vs0) | ((s0 == vs0) & (id0 <= ic0))) & (lv < nc0)
        s1m = ((s1 > vs1) | ((s1 == vs1) & (id1 <= ic1))) & (lv < nc1)
        plsc.store_scatter(out_v, [rix0, id0], onesf, mask=s0m)
        plsc.store_scatter(out_v, [rix1, id1], onesf, mask=s1m)
        return c

    lax.fori_loop(0, nv2s, sel_body, np.int32(0))
    pltpu.sync_copy(out_v, out_hbm.at[pl.ds(base, _RPW)])


@functools.partial(
    pl.kernel,
    out_type=jax.ShapeDtypeStruct((_B, _N), jnp.float32),
    mesh=plsc.VectorSubcoreMesh(
        core_axis_name="c", subcore_axis_name="s",
        num_cores=_NC, num_subcores=_NS),
    scratch_types=[
        pltpu.VMEM((_RPW, _N), jnp.float32),
        pltpu.VMEM((_RPW, _N), jnp.float32),
        pltpu.VMEM((512,), jnp.int32),
        pltpu.VMEM((2 * _GI,), jnp.int32),
        pltpu.VMEM((2 * _CAND,), jnp.int32),
        pltpu.VMEM((2 * _CAND,), jnp.int32),
        pltpu.SemaphoreType.DMA,
    ],
    compiler_params=pltpu.CompilerParams(needs_layout_passes=False),
)
def _topk_onehot(logits_hbm, out_hbm, rows_v, out_v, mx_v, gi_v, ck_v, ci_v,
                 sem):
    _topk_body(logits_hbm, out_hbm, rows_v, out_v, mx_v, gi_v, ck_v, ci_v,
               sem)


def kernel(logits, k):
    del k
    return _topk_onehot(logits)


# speedup vs baseline: 1.0544x; 1.0052x over previous
import functools

import jax
import jax.numpy as jnp
import numpy as np
from jax import lax
from jax.experimental import pallas as pl
from jax.experimental.pallas import tpu as pltpu
from jax.experimental.pallas import tpu_sc as plsc

_B = 64
_N = 8192
_K = 64
_L = 16
_NV = _N // _L
_NC = 2
_NS = 16
_RPW = _B // (_NC * _NS)
_UNROLL = 4

_SIGN = np.int32(-2147483648)
_MANT = np.int32(0x7FFFFFFF)
_ONE = np.int32(1)
_CAND = _N + 4 * _L
_GI = 272


def _monotone_keys(x):
    b = lax.bitcast_convert_type(x, jnp.int32)
    return b ^ ((b >> 31) & _MANT)


def _popcnt(m):
    return plsc.all_reduce_population_count(m)[0]


def _topk_body(logits_hbm, out_hbm, rows_v, out_v, mx_v, gi_v, ck_v, ci_v,
               sem):
    cid = lax.axis_index("c")
    sid = lax.axis_index("s")
    wid = sid * _NC + cid
    base = wid * _RPW
    pltpu.sync_copy(logits_hbm.at[pl.ds(base, _RPW)], rows_v)

    zeros = jnp.zeros((_L,), jnp.int32)
    zf = jnp.zeros((_L,), jnp.float32)
    minv = jnp.full((_L,), -jnp.inf, jnp.float32)

    def gmax_body(c, carry):
        def inner(i, ms):
            m0, m1 = ms
            for u in range(_UNROLL):
                sl = pl.ds((c * 32 + i * _UNROLL + u) * _L, _L)
                m0 = jnp.maximum(m0, rows_v[0, sl])
                m1 = jnp.maximum(m1, rows_v[1, sl])
                out_v[0, sl] = zf
                out_v[1, sl] = zf
            return m0, m1

        m0, m1 = lax.fori_loop(0, 32 // _UNROLL, inner, (minv, minv))
        mx_v[pl.ds(c * _L, _L)] = _monotone_keys(m0 + 0.0)
        mx_v[pl.ds(256 + c * _L, _L)] = _monotone_keys(m1 + 0.0)
        return carry

    lax.fori_loop(0, 16, gmax_body, np.int32(0))

    def mbit_body(j, tbs):
        tb0, tb1 = tbs
        bit = _ONE << (np.int32(31) - j)
        c0s = (tb0 | bit) ^ _SIGN
        c1s = (tb1 | bit) ^ _SIGN

        def cnt_body(i, accs):
            a0, a1 = accs
            for u in range(_UNROLL):
                sl = pl.ds((i * _UNROLL + u) * _L, _L)
                slb = pl.ds(256 + (i * _UNROLL + u) * _L, _L)
                a0 = a0 + (mx_v[sl] >= c0s).astype(jnp.int32)
                a1 = a1 + (mx_v[slb] >= c1s).astype(jnp.int32)
            return a0, a1

        a0, a1 = lax.fori_loop(0, 16 // _UNROLL, cnt_body, (zeros, zeros))
        tb0 = jnp.where(jnp.sum(a0) >= _K, tb0 | bit, tb0)
        tb1 = jnp.where(jnp.sum(a1) >= _K, tb1 | bit, tb1)
        return tb0, tb1

    tb0, tb1 = lax.fori_loop(0, 16, mbit_body,
                             (np.int32(0), np.int32(0)))

    ts0 = tb0 ^ _SIGN
    ts1 = tb1 ^ _SIGN
    iota = jnp.arange(_L, dtype=jnp.int32)

    def gl_body(i, goffs):
        g0, g1 = goffs
        gid = iota + i * _L
        m0 = mx_v[pl.ds(i * _L, _L)] >= ts0
        m1 = mx_v[pl.ds(256 + i * _L, _L)] >= ts1
        plsc.store_compressed(gi_v.at[pl.ds(g0, _L)], gid, mask=m0)
        plsc.store_compressed(gi_v.at[pl.ds(_GI + g1, _L)], gid, mask=m1)
        return g0 + _popcnt(m0), g1 + _popcnt(m1)

    ng0, ng1 = lax.fori_loop(0, 16, gl_body, (np.int32(0), np.int32(0)))
    gi_v[pl.ds(ng0, _L)] = zeros
    gi_v[pl.ds(_GI + ng1, _L)] = zeros

    rix0 = jnp.full((_L,), np.int32(0), jnp.int32)
    rix1 = jnp.full((_L,), np.int32(1), jnp.int32)
    nb0 = (ng0 + _L - 1) // _L
    nb1 = (ng1 + _L - 1) // _L
    nb = jnp.maximum(nb0, nb1)

    def comp_body(b, offs):
        o0, o1 = offs
        gv0 = gi_v[pl.ds(b * _L, _L)] & np.int32(255)
        gv1 = gi_v[pl.ds(_GI + b * _L, _L)] & np.int32(255)
        ok0 = (iota + b * _L) < ng0
        ok1 = (iota + b * _L) < ng1
        bp0 = (gv0 >> 4) * 512 + (gv0 & 15)
        bp1 = (gv1 >> 4) * 512 + (gv1 & 15)
        for i in range(32):
            p0 = bp0 + i * _L
            s0 = _monotone_keys(plsc.load_gather(rows_v, [rix0, p0]))
            m0 = (s0 >= ts0) & ok0
            plsc.store_compressed(ck_v.at[pl.ds(o0, _L)], s0, mask=m0)
            plsc.store_compressed(ci_v.at[pl.ds(o0, _L)], p0, mask=m0)
            o0 = o0 + _popcnt(m0)
            p1 = bp1 + i * _L
            s1 = _monotone_keys(plsc.load_gather(rows_v, [rix1, p1]))
            m1 = (s1 >= ts1) & ok1
            plsc.store_compressed(ck_v.at[pl.ds(_CAND + o1, _L)], s1,
                                  mask=m1)
            plsc.store_compressed(ci_v.at[pl.ds(_CAND + o1, _L)], p1,
                                  mask=m1)
            o1 = o1 + _popcnt(m1)
        return o0, o1

    nc0, nc1 = lax.fori_loop(0, nb, comp_body, (np.int32(0), np.int32(0)))

    nv2 = (jnp.maximum(nc0, nc1) + 4 * _L - 1) // (4 * _L)

    def lanes(i, u):
        return iota + (i * _UNROLL + u) * _L

    def bit2_body(j, tbs):
        t0, t1 = tbs
        bit = _ONE << (np.int32(31) - j)
        cb0 = t0 | bit
        cb1 = t1 | bit
        cs0 = cb0 ^ _SIGN
        cs1 = cb1 ^ _SIGN

        def cnt_body(i, accs):
            a0, a1 = accs
            for u in range(_UNROLL):
                lv = lanes(i, u)
                sl0 = pl.ds((i * _UNROLL + u) * _L, _L)
                sl1 = pl.ds(_CAND + (i * _UNROLL + u) * _L, _L)
                a0 = a0 + plsc.all_reduce_population_count(
                    (ck_v[sl0] >= cs0) & (lv < nc0))
                a1 = a1 + plsc.all_reduce_population_count(
                    (ck_v[sl1] >= cs1) & (lv < nc1))
            return a0, a1

        a0, a1 = lax.fori_loop(0, nv2, cnt_body, (zeros, zeros))
        t0 = jnp.where(a0[0] >= _K, cb0, t0)
        t1 = jnp.where(a1[0] >= _K, cb1, t1)
        return t0, t1

    t0, t1 = lax.fori_loop(0, 32, bit2_body, (np.int32(0), np.int32(0)))
    vs0 = t0 ^ _SIGN
    vs1 = t1 ^ _SIGN

    def cnt2_body(i, accs):
        g0, e0, g1, e1 = accs
        for u in range(_UNROLL):
            lv = lanes(i, u)
            s0 = ck_v[pl.ds((i * _UNROLL + u) * _L, _L)]
            s1 = ck_v[pl.ds(_CAND + (i * _UNROLL + u) * _L, _L)]
            ok0 = lv < nc0
            ok1 = lv < nc1
            g0 = g0 + plsc.all_reduce_population_count((s0 > vs0) & ok0)
            e0 = e0 + plsc.all_reduce_population_count((s0 == vs0) & ok0)
            g1 = g1 + plsc.all_reduce_population_count((s1 > vs1) & ok1)
            e1 = e1 + plsc.all_reduce_population_count((s1 == vs1) & ok1)
        return g0, e0, g1, e1

    g0, e0, g1, e1 = lax.fori_loop(0, nv2, cnt2_body,
                                   (zeros, zeros, zeros, zeros))
    need0 = _K - g0[0]
    need1 = _K - g1[0]
    extra0 = e0[0] - need0
    extra1 = e1[0] - need1
    m10 = extra0 + _ONE
    m11 = extra1 + _ONE

    def ibit_body(jj, ts_):
        i0, i1 = ts_
        bit = _ONE << (np.int32(12) - jj)
        c0 = i0 | bit
        c1 = i1 | bit

        def cnt_body(i, accs):
            a0, a1 = accs
            for u in range(_UNROLL):
                lv = lanes(i, u)
                sl0 = pl.ds((i * _UNROLL + u) * _L, _L)
                sl1 = pl.ds(_CAND + (i * _UNROLL + u) * _L, _L)
                a0 = a0 + plsc.all_reduce_population_count(
                    (ck_v[sl0] == vs0) & (ci_v[sl0] >= c0) & (lv < nc0))
                a1 = a1 + plsc.all_reduce_population_count(
                    (ck_v[sl1] == vs1) & (ci_v[sl1] >= c1) & (lv < nc1))
            return a0, a1

        a0, a1 = lax.fori_loop(0, nv2, cnt_body, (zeros, zeros))
        i0n = jnp.where(a0[0] >= m10, c0, i0)
        i1n = jnp.where(a1[0] >= m11, c1, i1)
        return (jnp.where(extra0 > 0, i0n, np.int32(8191)),
                jnp.where(extra1 > 0, i1n, np.int32(8191)))

    anyx = (extra0 > 0) | (extra1 > 0)
    ic0, ic1 = lax.fori_loop(
        0, jnp.where(anyx, np.int32(13), np.int32(0)), ibit_body,
        (jnp.where(extra0 > 0, np.int32(0), np.int32(8191)),
         jnp.where(extra1 > 0, np.int32(0), np.int32(8191))))

    onesf = jnp.ones((_L,), jnp.float32)
    nv2s = (jnp.maximum(nc0, nc1) + _L - 1) // _L

    def sel_body(i, c):
        lv = iota + i * _L
        sl0 = pl.ds(i * _L, _L)
        sl1 = pl.ds(_CAND + i * _L, _L)
        s0 = ck_v[sl0]
        s1 = ck_v[sl1]
        id0 = ci_v[sl0]
        id1 = ci_v[sl1]
        s0m = ((s0 > vs0) | ((s0 == vs0) & (id0 <= ic0))) & (lv < nc0)
        s1m = ((s1 > vs1) | ((s1 == vs1) & (id1 <= ic1))) & (lv < nc1)
        plsc.store_scatter(out_v, [rix0, id0], onesf, mask=s0m)
        plsc.store_scatter(out_v, [rix1, id1], onesf, mask=s1m)
        return c

    lax.fori_loop(0, nv2s, sel_body, np.int32(0))
    pltpu.sync_copy(out_v, out_hbm.at[pl.ds(base, _RPW)])


@functools.partial(
    pl.kernel,
    out_type=jax.ShapeDtypeStruct((_B, _N), jnp.float32),
    mesh=plsc.VectorSubcoreMesh(
        core_axis_name="c", subcore_axis_name="s",
        num_cores=_NC, num_subcores=_NS),
    scratch_types=[
        pltpu.VMEM((_RPW, _N), jnp.float32),
        pltpu.VMEM((_RPW, _N), jnp.float32),
        pltpu.VMEM((512,), jnp.int32),
        pltpu.VMEM((2 * _GI,), jnp.int32),
        pltpu.VMEM((2 * _CAND,), jnp.int32),
        pltpu.VMEM((2 * _CAND,), jnp.int32),
        pltpu.SemaphoreType.DMA,
    ],
    compiler_params=pltpu.CompilerParams(needs_layout_passes=False),
)
def _topk_onehot(logits_hbm, out_hbm, rows_v, out_v, mx_v, gi_v, ck_v, ci_v,
                 sem):
    _topk_body(logits_hbm, out_hbm, rows_v, out_v, mx_v, gi_v, ck_v, ci_v,
               sem)


def kernel(logits, k):
    del k
    return _topk_onehot(logits)
